# baseline (speedup 1.0000x reference)
import functools

import jax
import jax.numpy as jnp
from jax import lax
from jax.experimental import pallas as pl
from jax.experimental.pallas import tpu as pltpu
from jax.experimental.pallas import tpu_sc as plsc

_LOG1P_C = (
    -1.74140775e-02, 8.26912371e-02, -1.90354337e-01, 3.15747317e-01,
    -4.97373216e-01, 9.99847697e-01, 1.47206501e-06,
)


def _log1p_poly(w):
    acc = jnp.full_like(w, _LOG1P_C[0])
    for c in _LOG1P_C[1:]:
        acc = acc * w + jnp.float32(c)
    return acc


def _sc_conf(x0, x1, y):
    B, N = y.shape
    info = plsc.get_sparse_core_info()
    nw = info.num_cores * info.num_subcores
    rpw = B // nw
    assert B % nw == 0 and N % 32 == 0
    nchunks2 = N // 32
    nchunks = N // 16
    mesh = plsc.VectorSubcoreMesh(core_axis_name="c", subcore_axis_name="s")

    @functools.partial(
        pl.kernel,
        mesh=mesh,
        compiler_params=pltpu.CompilerParams(needs_layout_passes=False),
        out_type=jax.ShapeDtypeStruct((nw, 16), jnp.float32),
        scratch_types=[
            pltpu.VMEM((N,), jnp.float32),
            pltpu.VMEM((N,), jnp.float32),
            pltpu.VMEM((N,), jnp.float32),
            pltpu.VMEM((N,), jnp.float32),
            pltpu.VMEM((N,), jnp.int32),
            pltpu.VMEM((16,), jnp.float32),
            pltpu.SemaphoreType.DMA,
            pltpu.SemaphoreType.DMA,
        ],
    )
    def body(x0h, x1h, yh, outh, va0, vb0, va1, vb1, vy, vout, sem0, sem1):
        wid = lax.axis_index("s") * info.num_cores + lax.axis_index("c")
        lane = lax.broadcasted_iota(jnp.int32, (16,), 0)
        res = jnp.zeros((16,), jnp.float32)
        zf = jnp.zeros((16,), jnp.float32)
        abufs = (va0, va1)
        bbufs = (vb0, vb1)
        sems = (sem0, sem1)
        r0 = wid * rpw
        pend = (pltpu.async_copy(x0h.at[r0], va0, sem0),
                pltpu.async_copy(x1h.at[r0], vb0, sem0))
        for rl in range(rpw):
            va = abufs[rl % 2]
            vb = bbufs[rl % 2]
            pltpu.sync_copy(yh.at[r0 + rl], vy)
            for h in pend:
                h.wait()
            if rl + 1 < rpw:
                nb = (rl + 1) % 2
                pend = (
                    pltpu.async_copy(x0h.at[r0 + rl + 1], abufs[nb], sems[nb]),
                    pltpu.async_copy(x1h.at[r0 + rl + 1], bbufs[nb], sems[nb]))

            def main_it(i, carry):
                npv, cev = carry
                for q in range(2):
                    sl = pl.ds(i * 32 + q * 16, 16)
                    a0 = va[sl]
                    a1 = vb[sl]
                    yy = vy[sl]
                    t = a0 - a1
                    u = jnp.where(yy > 0, t, -t)
                    au = jnp.abs(t)
                    ru = jnp.maximum(u, jnp.float32(0.0))
                    ce = ru + _log1p_poly(jnp.exp(-au))
                    cev = cev + ce
                    npv = npv + jnp.where(
                        yy > 0, jnp.float32(1.0), jnp.float32(0.0))
                return npv, cev

            npv, cev = lax.fori_loop(0, nchunks2, main_it, (zf, zf))
            np_s = jnp.sum(npv)
            ce_s = jnp.sum(cev)
            k = jnp.minimum(jnp.float32(3.0) * np_s, jnp.float32(N - 1))
            j = jnp.float32(N) - k

            def tail_zero():
                return jnp.float32(0.0)

            def tail_bisect():
                def mk(i, acc):
                    for q in range(2):
                        sl = pl.ds(i * 32 + q * 16, 16)
                        a0 = va[sl]
                        a1 = vb[sl]
                        yy = vy[sl]
                        t = a0 - a1
                        u = jnp.where(yy > 0, t, -t)
                        au = jnp.abs(t)
                        ru = jnp.maximum(u, jnp.float32(0.0))
                        ce = ru + _log1p_poly(jnp.exp(-au))
                        va[sl] = jnp.where(yy > 0, jnp.float32(0.0), ce)
                    return acc

                lax.fori_loop(0, nchunks2, mk, jnp.int32(0))

                def bis(_, lohi):
                    lo, hi = lohi
                    mid = lo + (hi - lo) // 2

                    def cnt(i, acc):
                        kv = plsc.bitcast(va[pl.ds(i * 16, 16)], jnp.int32)
                        return acc + jnp.where(
                            kv < mid, jnp.float32(1.0), jnp.float32(0.0))

                    c = jnp.sum(lax.fori_loop(0, nchunks, cnt, zf))
                    take_hi = c < j
                    return (jnp.where(take_hi, mid, lo),
                            jnp.where(take_hi, hi, mid))

                lo, _ = lax.fori_loop(
                    0, 31, bis, (jnp.int32(0), jnp.int32(0x7F800001)))

                def fin(i, acc):
                    sb, cb = acc
                    mv = va[pl.ds(i * 16, 16)]
                    kv = plsc.bitcast(mv, jnp.int32)
                    ltm = kv < lo
                    sb = sb + jnp.where(ltm, mv, jnp.float32(0.0))
                    cb = cb + jnp.where(ltm, jnp.float32(1.0), jnp.float32(0.0))
                    return sb, cb

                sbv, cbv = lax.fori_loop(0, nchunks, fin, (zf, zf))
                vj = lax.bitcast_convert_type(lo, jnp.float32)
                return jnp.sum(sbv) + (j - jnp.sum(cbv)) * vj

            ss = lax.cond(j <= np_s, tail_zero, tail_bisect)
            contrib = ce_s - ss
            res = jnp.where(lane == rl, contrib, res)
            res = jnp.where(lane == rpw + rl, np_s, res)
        vout[...] = res
        pltpu.sync_copy(vout, outh.at[wid])

    return body(x0, x1, y)


def _sl1(x):
    ax = jnp.abs(x)
    return jnp.where(ax < 1.0, 0.5 * x * x, ax - 0.5)


def _tc_masked_sums(dlT, dmT, y):
    _, B, N = dlT.shape
    gb = B // 32
    grid = (gb,)

    def body(dl_ref, dm_ref, y_ref, ol_ref, om_ref):
        i = pl.program_id(0)
        posm = (y_ref[...] > 0).astype(jnp.float32)
        accl = jnp.zeros_like(posm)
        for d in range(4):
            accl = accl + _sl1(dl_ref[d].astype(jnp.float32))
        ol_ref[0, i] = jnp.sum(accl * posm)
        accm = jnp.zeros_like(posm)
        for d in range(10):
            accm = accm + _sl1(dm_ref[d].astype(jnp.float32))
        om_ref[0, i] = jnp.sum(accm * posm)

    return pl.pallas_call(
        body,
        grid=grid,
        in_specs=[
            pl.BlockSpec((4, 32, N), lambda i: (0, i, 0)),
            pl.BlockSpec((10, 32, N), lambda i: (0, i, 0)),
            pl.BlockSpec((32, N), lambda i: (i, 0)),
        ],
        out_specs=[
            pl.BlockSpec((1, gb), lambda i: (0, 0), memory_space=pltpu.SMEM),
            pl.BlockSpec((1, gb), lambda i: (0, 0), memory_space=pltpu.SMEM),
        ],
        out_shape=[
            jax.ShapeDtypeStruct((1, gb), jnp.float32),
            jax.ShapeDtypeStruct((1, gb), jnp.float32),
        ],
        compiler_params=pltpu.CompilerParams(
            dimension_semantics=("arbitrary",)),
    )(dlT, dmT, y)


def kernel(loc_preds, loc_targets, conf_preds, conf_targets,
           landmarks_preds, landmarks_targets):
    B, N = conf_targets.shape
    x0 = conf_preds[:, :, 0]
    x1 = conf_preds[:, :, 1]
    sc_out = _sc_conf(x0, x1, conf_targets)

    dlT = (jnp.moveaxis(loc_preds, 2, 0)
           - jnp.moveaxis(loc_targets, 2, 0)).astype(jnp.float8_e4m3fn)
    dmT = (jnp.moveaxis(landmarks_preds, 2, 0)
           - jnp.moveaxis(landmarks_targets, 2, 0)).astype(jnp.float8_e4m3fn)
    loc_p, lm_p = _tc_masked_sums(dlT, dmT, conf_targets)

    info = plsc.get_sparse_core_info()
    rpw = B // (info.num_cores * info.num_subcores)
    conf_sum = jnp.sum(sc_out[:, :rpw])
    num_matched = jnp.sum(sc_out[:, rpw:2 * rpw])
    return (jnp.sum(loc_p) + jnp.sum(lm_p) + conf_sum) / num_matched


# speedup vs baseline: 1.3827x; 1.3827x over previous
import functools

import jax
import jax.numpy as jnp
from jax import lax
from jax.experimental import pallas as pl
from jax.experimental.pallas import tpu as pltpu
from jax.experimental.pallas import tpu_sc as plsc

_LOG1P_C = (
    -1.74140775e-02, 8.26912371e-02, -1.90354337e-01, 3.15747317e-01,
    -4.97373216e-01, 9.99847697e-01, 1.47206501e-06,
)


def _log1p_poly(w):
    acc = jnp.full_like(w, _LOG1P_C[0])
    for c in _LOG1P_C[1:]:
        acc = acc * w + jnp.float32(c)
    return acc


def _sc_conf(x0, x1, y):
    B, N = y.shape
    info = plsc.get_sparse_core_info()
    nw = info.num_cores * info.num_subcores
    rpw = B // nw
    assert B % nw == 0 and N % 32 == 0
    nchunks2 = N // 32
    nchunks = N // 16
    mesh = plsc.VectorSubcoreMesh(core_axis_name="c", subcore_axis_name="s")

    @functools.partial(
        pl.kernel,
        mesh=mesh,
        compiler_params=pltpu.CompilerParams(needs_layout_passes=False),
        out_type=jax.ShapeDtypeStruct((nw, 16), jnp.float32),
        scratch_types=[
            pltpu.VMEM((N,), jnp.float32),
            pltpu.VMEM((N,), jnp.float32),
            pltpu.VMEM((N,), jnp.float32),
            pltpu.VMEM((N,), jnp.float32),
            pltpu.VMEM((N,), jnp.int32),
            pltpu.VMEM((16,), jnp.float32),
            pltpu.SemaphoreType.DMA,
            pltpu.SemaphoreType.DMA,
        ],
    )
    def body(x0h, x1h, yh, outh, va0, vb0, va1, vb1, vy, vout, sem0, sem1):
        wid = lax.axis_index("s") * info.num_cores + lax.axis_index("c")
        lane = lax.broadcasted_iota(jnp.int32, (16,), 0)
        res = jnp.zeros((16,), jnp.float32)
        zf = jnp.zeros((16,), jnp.float32)
        abufs = (va0, va1)
        bbufs = (vb0, vb1)
        sems = (sem0, sem1)
        r0 = wid * rpw
        pend = (pltpu.async_copy(x0h.at[r0], va0, sem0),
                pltpu.async_copy(x1h.at[r0], vb0, sem0))
        for rl in range(rpw):
            va = abufs[rl % 2]
            vb = bbufs[rl % 2]
            pltpu.sync_copy(yh.at[r0 + rl], vy)
            for h in pend:
                h.wait()
            if rl + 1 < rpw:
                nb = (rl + 1) % 2
                pend = (
                    pltpu.async_copy(x0h.at[r0 + rl + 1], abufs[nb], sems[nb]),
                    pltpu.async_copy(x1h.at[r0 + rl + 1], bbufs[nb], sems[nb]))

            def main_it(i, carry):
                npv, cev = carry
                for q in range(2):
                    sl = pl.ds(i * 32 + q * 16, 16)
                    a0 = va[sl]
                    a1 = vb[sl]
                    yy = vy[sl]
                    t = a0 - a1
                    u = jnp.where(yy > 0, t, -t)
                    au = jnp.abs(t)
                    ru = jnp.maximum(u, jnp.float32(0.0))
                    ce = ru + _log1p_poly(jnp.exp(-au))
                    cev = cev + ce
                    npv = npv + jnp.where(
                        yy > 0, jnp.float32(1.0), jnp.float32(0.0))
                return npv, cev

            npv, cev = lax.fori_loop(0, nchunks2, main_it, (zf, zf))
            np_s = jnp.sum(npv)
            ce_s = jnp.sum(cev)
            k = jnp.minimum(jnp.float32(3.0) * np_s, jnp.float32(N - 1))
            j = jnp.float32(N) - k

            def tail_zero():
                return jnp.float32(0.0)

            def tail_bisect():
                def mk(i, acc):
                    for q in range(2):
                        sl = pl.ds(i * 32 + q * 16, 16)
                        a0 = va[sl]
                        a1 = vb[sl]
                        yy = vy[sl]
                        t = a0 - a1
                        u = jnp.where(yy > 0, t, -t)
                        au = jnp.abs(t)
                        ru = jnp.maximum(u, jnp.float32(0.0))
                        ce = ru + _log1p_poly(jnp.exp(-au))
                        va[sl] = jnp.where(yy > 0, jnp.float32(0.0), ce)
                    return acc

                lax.fori_loop(0, nchunks2, mk, jnp.int32(0))

                def bis(_, lohi):
                    lo, hi = lohi
                    mid = lo + (hi - lo) // 2

                    def cnt(i, acc):
                        kv = plsc.bitcast(va[pl.ds(i * 16, 16)], jnp.int32)
                        return acc + jnp.where(
                            kv < mid, jnp.float32(1.0), jnp.float32(0.0))

                    c = jnp.sum(lax.fori_loop(0, nchunks, cnt, zf))
                    take_hi = c < j
                    return (jnp.where(take_hi, mid, lo),
                            jnp.where(take_hi, hi, mid))

                lo, _ = lax.fori_loop(
                    0, 31, bis, (jnp.int32(0), jnp.int32(0x7F800001)))

                def fin(i, acc):
                    sb, cb = acc
                    mv = va[pl.ds(i * 16, 16)]
                    kv = plsc.bitcast(mv, jnp.int32)
                    ltm = kv < lo
                    sb = sb + jnp.where(ltm, mv, jnp.float32(0.0))
                    cb = cb + jnp.where(ltm, jnp.float32(1.0), jnp.float32(0.0))
                    return sb, cb

                sbv, cbv = lax.fori_loop(0, nchunks, fin, (zf, zf))
                vj = lax.bitcast_convert_type(lo, jnp.float32)
                return jnp.sum(sbv) + (j - jnp.sum(cbv)) * vj

            ss = lax.cond(j <= np_s, tail_zero, tail_bisect)
            contrib = ce_s - ss
            res = jnp.where(lane == rl, contrib, res)
            res = jnp.where(lane == rpw + rl, np_s, res)
        vout[...] = res
        pltpu.sync_copy(vout, outh.at[wid])

    return body(x0, x1, y)


def _sl1(x):
    ax = jnp.abs(x)
    return jnp.where(ax < 1.0, 0.5 * x * x, ax - 0.5)


def _tc_masked_sums(dlT, dmT, y):
    _, B, N = dlT.shape
    gb = B // 16
    grid = (gb,)

    def body(dl_ref, dm_ref, y_ref, ol_ref, om_ref):
        i = pl.program_id(0)
        posm = (y_ref[...] > 0).astype(jnp.float32)
        accl = jnp.zeros_like(posm)
        for d in range(4):
            accl = accl + _sl1(dl_ref[d].astype(jnp.float32))
        ol_ref[0, i] = jnp.sum(accl * posm)
        accm = jnp.zeros_like(posm)
        for d in range(10):
            accm = accm + _sl1(dm_ref[d].astype(jnp.float32))
        om_ref[0, i] = jnp.sum(accm * posm)

    return pl.pallas_call(
        body,
        grid=grid,
        in_specs=[
            pl.BlockSpec((4, 16, N), lambda i: (0, i, 0)),
            pl.BlockSpec((10, 16, N), lambda i: (0, i, 0)),
            pl.BlockSpec((16, N), lambda i: (i, 0)),
        ],
        out_specs=[
            pl.BlockSpec((1, gb), lambda i: (0, 0), memory_space=pltpu.SMEM),
            pl.BlockSpec((1, gb), lambda i: (0, 0), memory_space=pltpu.SMEM),
        ],
        out_shape=[
            jax.ShapeDtypeStruct((1, gb), jnp.float32),
            jax.ShapeDtypeStruct((1, gb), jnp.float32),
        ],
        compiler_params=pltpu.CompilerParams(
            dimension_semantics=("arbitrary",)),
    )(dlT, dmT, y)


def kernel(loc_preds, loc_targets, conf_preds, conf_targets,
           landmarks_preds, landmarks_targets):
    B, N = conf_targets.shape
    x0 = conf_preds[:, :, 0]
    x1 = conf_preds[:, :, 1]
    sc_out = _sc_conf(x0, x1, conf_targets)

    dlT = (jnp.moveaxis(loc_preds, 2, 0)
           - jnp.moveaxis(loc_targets, 2, 0)).astype(jnp.bfloat16)
    dmT = (jnp.moveaxis(landmarks_preds, 2, 0)
           - jnp.moveaxis(landmarks_targets, 2, 0)).astype(jnp.bfloat16)
    loc_p, lm_p = _tc_masked_sums(dlT, dmT, conf_targets)

    info = plsc.get_sparse_core_info()
    rpw = B // (info.num_cores * info.num_subcores)
    conf_sum = jnp.sum(sc_out[:, :rpw])
    num_matched = jnp.sum(sc_out[:, rpw:2 * rpw])
    return (jnp.sum(loc_p) + jnp.sum(lm_p) + conf_sum) / num_matched
